# baseline (speedup 1.0000x reference)
import functools

import jax
import jax.numpy as jnp
from jax import lax
from jax.experimental import pallas as pl
from jax.experimental.pallas import tpu as pltpu
from jax.experimental.pallas import tpu_sc as plsc

VOCAB = 1000000
EMB = 64
PADW = 128
NC = 2
NS = 16
NW = NC * NS
NI = 4096
NJ = 200
WB = NI // NW
L = 16


def _gather_kernel(idxf_hbm, idxr_hbm, tab_hbm, out_hbm, idxf_v, idxr_v,
                   buf_a, buf_b, st_a, st_b, sem_a, sem_b):
    c = lax.axis_index("c")
    s = lax.axis_index("s")
    w = s * NC + c
    col0 = w * WB
    pltpu.sync_copy(idxf_hbm.at[pl.ds(0, NJ), pl.ds(col0, WB)], idxf_v)
    pltpu.sync_copy(idxr_hbm.at[pl.ds(0, NJ), pl.ds(col0, WB)], idxr_v)

    def fire(j, buf, sem):
        pltpu.async_copy(tab_hbm.at[idxf_v.at[j]], buf, sem)

    def drain(buf, sem):
        pltpu.make_async_copy(tab_hbm.at[idxf_v.at[0]], buf, sem).wait()

    def transpose(j, buf, st):
        for b in range(WB // L):
            rows = lax.iota(jnp.int32, L) + (b * L)
            raw = idxr_v[j, pl.ds(b * L, L)]
            hoff = (raw & 1) * EMB

            def col_body(ci, carry, rows=rows, hoff=hoff, b=b):
                vals = plsc.load_gather(buf, [rows, hoff + ci])
                st[ci, pl.ds(b * L, L)] = vals
                return carry

            lax.fori_loop(0, EMB, col_body, 0)

    def writeback(j, st):
        pltpu.sync_copy(st, out_hbm.at[j, :, pl.ds(col0, WB)])

    fire(0, buf_a, sem_a)

    def body(p, carry):
        j = 2 * p
        drain(buf_a, sem_a)
        fire(j + 1, buf_b, sem_b)
        transpose(j, buf_a, st_a)
        writeback(j, st_a)
        drain(buf_b, sem_b)
        fire(j + 2, buf_a, sem_a)
        transpose(j + 1, buf_b, st_b)
        writeback(j + 1, st_b)
        return carry

    lax.fori_loop(0, NJ // 2 - 1, body, 0)

    j = NJ - 2
    drain(buf_a, sem_a)
    fire(j + 1, buf_b, sem_b)
    transpose(j, buf_a, st_a)
    writeback(j, st_a)
    drain(buf_b, sem_b)
    transpose(j + 1, buf_b, st_b)
    writeback(j + 1, st_b)


@jax.jit
def _run(idxf, idxr, tabf):
    mesh = plsc.VectorSubcoreMesh(core_axis_name="c", subcore_axis_name="s")
    k = functools.partial(
        pl.kernel,
        mesh=mesh,
        out_type=jax.ShapeDtypeStruct((NJ, EMB, NI), jnp.float32),
        scratch_types=[
            pltpu.VMEM((NJ, WB), jnp.int32),
            pltpu.VMEM((NJ, WB), jnp.int32),
            pltpu.VMEM((WB, PADW), jnp.float32),
            pltpu.VMEM((WB, PADW), jnp.float32),
            pltpu.VMEM((EMB, WB), jnp.float32),
            pltpu.VMEM((EMB, WB), jnp.float32),
            pltpu.SemaphoreType.DMA,
            pltpu.SemaphoreType.DMA,
        ],
        compiler_params=pltpu.CompilerParams(needs_layout_passes=False),
    )(_gather_kernel)
    return k(idxf, idxr, tabf)


def kernel(data, tvectors):
    di = data.astype(jnp.int32)
    idxf = (di >> 1).T
    idxr = di.T
    tabf = tvectors.reshape(VOCAB // 2, PADW)
    out = _run(idxf, idxr, tabf)
    return jnp.transpose(out, (2, 0, 1))


# speedup vs baseline: 1.5512x; 1.5512x over previous
import functools

import jax
import jax.numpy as jnp
from jax import lax
from jax.experimental import pallas as pl
from jax.experimental.pallas import tpu as pltpu
from jax.experimental.pallas import tpu_sc as plsc

VOCAB = 1000000
EMB = 64
NC = 2
NS = 16
NW = NC * NS
B = 4096 * 200
B_PER_W = B // NW
CHUNK = 512
KK = 1
GROUP = KK * CHUNK
NGROUPS = B_PER_W // GROUP
NCHUNKS_W = B_PER_W // CHUNK


def _gather_kernel(idx_hbm, table_hbm, out_hbm, idx_v, rows_a, rows_b, sem_a, sem_b):
    c = lax.axis_index("c")
    s = lax.axis_index("s")
    wid = s * NC + c
    base = wid * B_PER_W
    pltpu.sync_copy(idx_hbm.at[wid], idx_v)

    def fire(g, buf, sem):
        for j in range(KK):
            pltpu.async_copy(
                table_hbm.at[idx_v.at[g * KK + j]],
                buf.at[pl.ds(j * CHUNK, CHUNK)],
                sem,
            )

    def drain(buf, sem):
        pltpu.make_async_copy(out_hbm.at[pl.ds(0, GROUP)], buf, sem).wait()

    def writeback(g, buf):
        pltpu.sync_copy(buf, out_hbm.at[pl.ds(base + g * GROUP, GROUP)])

    fire(0, rows_a, sem_a)

    def body(i, carry):
        g = 2 * i
        drain(rows_a, sem_a)
        fire(g + 1, rows_b, sem_b)
        writeback(g, rows_a)
        drain(rows_b, sem_b)
        fire(g + 2, rows_a, sem_a)
        writeback(g + 1, rows_b)
        return carry

    lax.fori_loop(0, NGROUPS // 2 - 1, body, 0)

    g = NGROUPS - 2
    drain(rows_a, sem_a)
    fire(g + 1, rows_b, sem_b)
    writeback(g, rows_a)
    drain(rows_b, sem_b)
    writeback(g + 1, rows_b)


@jax.jit
def _run(idx, tvectors):
    mesh = plsc.VectorSubcoreMesh(core_axis_name="c", subcore_axis_name="s")
    k = functools.partial(
        pl.kernel,
        mesh=mesh,
        out_type=jax.ShapeDtypeStruct((B, EMB), jnp.float32),
        scratch_types=[
            pltpu.VMEM((NCHUNKS_W, CHUNK), jnp.int32),
            pltpu.VMEM((GROUP, EMB), jnp.float32),
            pltpu.VMEM((GROUP, EMB), jnp.float32),
            pltpu.SemaphoreType.DMA,
            pltpu.SemaphoreType.DMA,
        ],
        compiler_params=pltpu.CompilerParams(use_tc_tiling_on_sc=False),
    )(_gather_kernel)
    return k(idx, tvectors)


def kernel(data, tvectors):
    idx = data.astype(jnp.int32).reshape(NW, NCHUNKS_W, CHUNK)
    out = _run(idx, tvectors)
    return out.reshape(4096, 200, EMB)
